# baseline (speedup 1.0000x reference)
import functools

import jax
import jax.numpy as jnp
from jax import lax
from jax.experimental import pallas as pl
from jax.experimental.pallas import tpu as pltpu
from jax.experimental.pallas import tpu_sc as plsc

DIM = 512
MEM = 100000
NCH = DIM // 16
NWORKERS = 32

SC_ROWS = 24576
NPW = SC_ROWS // NWORKERS
CHUNK = 48
NBUF = 2
NCHUNKS = NPW // CHUNK
GROUPS = CHUNK // 16
TC_BLOCK = 4000
TC_BLOCK0 = SC_ROWS // TC_BLOCK
TC_NB = (MEM - TC_BLOCK0 * TC_BLOCK) // TC_BLOCK


def _pmean_body(pattern_ref, out_ref):
    out_ref[...] = jnp.mean(pattern_ref[...], axis=0, keepdims=True)


def _tc_scan_body(pmean_ref, sp_ref, mean_ref, var_ref, w1_ref, b1_ref,
                  w2_ref, b2_ref, part_ref, min_ref):
    i = pl.program_id(0)

    @pl.when(i == 0)
    def _init():
        pmean = pmean_ref[...]
        z = jnp.mean(jnp.abs((pmean - mean_ref[...]) /
                             (jnp.sqrt(var_ref[...]) + 1e-6)))
        stat_sim = jnp.exp(-z * 0.5)
        h = (lax.dot_general(pmean, w1_ref[:, :DIM], (((1,), (1,)), ((), ())))
             + lax.dot_general(mean_ref[...], w1_ref[:, DIM:],
                               (((1,), (1,)), ((), ())))
             + b1_ref[...])
        h = jnp.maximum(h, 0.0)
        neural = jax.nn.sigmoid(jnp.sum(h * w2_ref[...]) + b2_ref[0])
        part_ref[0] = stat_sim * 0.4 + neural * 0.3
        min_ref[0] = jnp.inf

    diff = sp_ref[...] - pmean_ref[...]
    d2 = jnp.min(jnp.sum(diff * diff, axis=1))
    min_ref[0] = jnp.minimum(min_ref[0], d2)


def _combine_body(sc_min_ref, tc_min_ref, part_ref, out_ref):
    m = jnp.minimum(jnp.min(sc_min_ref[...]), tc_min_ref[0])
    out_ref[0] = part_ref[0] + 0.3 * jnp.exp(-jnp.sqrt(m))


def _sc_scan_body(sp_hbm, pm_hbm, out_hbm, pat_v, bufs, a_v, stage_v, sems):
    cid = lax.axis_index("c")
    sid = lax.axis_index("s")
    wid = sid * 2 + cid
    base = wid * NPW

    pltpu.sync_copy(pm_hbm, pat_v)
    pats = [pat_v[pl.ds(16 * c, 16)] for c in range(NCH)]
    lane = jax.lax.iota(jnp.int32, 16)

    def start(i, buf, sem):
        row0 = pl.multiple_of(base + i * CHUNK, 8)
        pltpu.make_async_copy(sp_hbm.at[pl.ds(row0, CHUNK)], buf, sem).start()

    def wait(buf, sem):
        pltpu.make_async_copy(sp_hbm.at[pl.ds(base, CHUNK)], buf, sem).wait()

    def compute(buf, mins):
        def group_body(g, mins):
            def row_body(q, carry):
                for u in range(4):
                    r = q * 4 + u
                    row = g * 16 + r
                    d = buf[row, pl.ds(0, 16)] - pats[0]
                    acc = d * d
                    for c in range(1, NCH):
                        d = buf[row, pl.ds(16 * c, 16)] - pats[c]
                        acc = acc + d * d
                    a_v[r] = acc
                return carry

            lax.fori_loop(0, 4, row_body, 0)
            tot = plsc.load_gather(
                a_v, [lane, jnp.full((16,), 0, jnp.int32)])
            for c in range(1, 16):
                tot = tot + plsc.load_gather(
                    a_v, [lane, jnp.full((16,), c, jnp.int32)])
            return jnp.minimum(mins, tot)

        return lax.fori_loop(0, GROUPS, group_body, mins)

    for b in range(NBUF):
        start(b, bufs[b], sems[b])
    mins0 = jnp.full((16,), jnp.inf, dtype=jnp.float32)

    def ring_body(jb, mins):
        for b in range(NBUF):
            i = jb * NBUF + b
            wait(bufs[b], sems[b])
            mins = compute(bufs[b], mins)

            @pl.when(i + NBUF < NCHUNKS)
            def _(i=i, b=b):
                start(i + NBUF, bufs[b], sems[b])

        return mins

    mins = lax.fori_loop(0, NCHUNKS // NBUF, ring_body, mins0)

    stage_v[...] = mins
    pltpu.sync_copy(stage_v, out_hbm.at[wid])


@functools.partial(
    pl.kernel,
    mesh=plsc.VectorSubcoreMesh(core_axis_name="c", subcore_axis_name="s"),
    out_type=jax.ShapeDtypeStruct((NWORKERS, 16), jnp.float32),
    compiler_params=pltpu.CompilerParams(needs_layout_passes=False),
    scratch_types=(
        [pltpu.VMEM((DIM,), jnp.float32)]
        + [pltpu.VMEM((CHUNK, DIM), jnp.float32) for _ in range(NBUF)]
        + [pltpu.VMEM((16, 16), jnp.float32),
           pltpu.VMEM((16,), jnp.float32)]
        + [pltpu.SemaphoreType.DMA for _ in range(NBUF)]
    ),
)
def _sc_scan(sp_hbm, pm_hbm, out_hbm, pat_v, *rest):
    bufs = rest[:NBUF]
    a_v, stage_v = rest[NBUF], rest[NBUF + 1]
    sems = rest[NBUF + 2:]
    _sc_scan_body(sp_hbm, pm_hbm, out_hbm, pat_v, bufs, a_v, stage_v, sems)


def kernel(pattern, self_patterns, self_mean, self_var, W1, b1, W2, b2):
    pmean = pl.pallas_call(
        _pmean_body,
        in_specs=[pl.BlockSpec((1024, DIM), lambda: (0, 0))],
        out_specs=pl.BlockSpec((1, DIM), lambda: (0, 0)),
        out_shape=jax.ShapeDtypeStruct((1, DIM), jnp.float32),
    )(pattern)

    sc_min = _sc_scan(self_patterns, pmean.reshape(DIM))

    part, tc_min = pl.pallas_call(
        _tc_scan_body,
        grid=(TC_NB,),
        in_specs=[
            pl.BlockSpec((1, DIM), lambda i: (0, 0)),
            pl.BlockSpec((TC_BLOCK, DIM), lambda i: (TC_BLOCK0 + i, 0)),
            pl.BlockSpec((1, DIM), lambda i: (0, 0)),
            pl.BlockSpec((1, DIM), lambda i: (0, 0)),
            pl.BlockSpec((DIM, 2 * DIM), lambda i: (0, 0)),
            pl.BlockSpec((1, DIM), lambda i: (0, 0)),
            pl.BlockSpec((1, DIM), lambda i: (0, 0)),
            pl.BlockSpec(memory_space=pltpu.SMEM),
        ],
        out_specs=[pl.BlockSpec(memory_space=pltpu.SMEM),
                   pl.BlockSpec(memory_space=pltpu.SMEM)],
        out_shape=[jax.ShapeDtypeStruct((1,), jnp.float32),
                   jax.ShapeDtypeStruct((1,), jnp.float32)],
    )(pmean, self_patterns, self_mean.reshape(1, DIM),
      self_var.reshape(1, DIM), W1, b1.reshape(1, DIM), W2, b2)

    m = jnp.minimum(jnp.min(sc_min), tc_min[0])
    return part[0] + 0.3 * jnp.exp(-jnp.sqrt(m))


# speedup vs baseline: 1.0241x; 1.0241x over previous
import functools

import jax
import jax.numpy as jnp
from jax import lax
from jax.experimental import pallas as pl
from jax.experimental.pallas import tpu as pltpu
from jax.experimental.pallas import tpu_sc as plsc

DIM = 512
MEM = 100000
NCH = DIM // 16
NWORKERS = 32
PATROWS = 1024
PROWS_PW = PATROWS // 16

SC_ROWS = 24576
NPW = SC_ROWS // NWORKERS
CHUNK = 48
NBUF = 2
NCHUNKS = NPW // CHUNK
GROUPS = CHUNK // 16
TC_BLOCK = 4000
TC_BLOCK0 = SC_ROWS // TC_BLOCK
TC_NB = (MEM - TC_BLOCK0 * TC_BLOCK) // TC_BLOCK


def _tc_scan_body(pattern_ref, sp_ref, mean_ref, var_ref, w1_ref, b1_ref,
                  w2_ref, b2_ref, part_ref, min_ref, pmean_ref):
    i = pl.program_id(0)

    @pl.when(i == 0)
    def _init():
        pmean = jnp.mean(pattern_ref[...], axis=0, keepdims=True)
        pmean_ref[...] = pmean
        z = jnp.mean(jnp.abs((pmean - mean_ref[...]) /
                             (jnp.sqrt(var_ref[...]) + 1e-6)))
        stat_sim = jnp.exp(-z * 0.5)
        h = (lax.dot_general(pmean, w1_ref[:, :DIM], (((1,), (1,)), ((), ())))
             + lax.dot_general(mean_ref[...], w1_ref[:, DIM:],
                               (((1,), (1,)), ((), ())))
             + b1_ref[...])
        h = jnp.maximum(h, 0.0)
        neural = jax.nn.sigmoid(jnp.sum(h * w2_ref[...]) + b2_ref[0])
        part_ref[0] = stat_sim * 0.4 + neural * 0.3
        min_ref[0] = jnp.inf

    diff = sp_ref[...] - pmean_ref[...]
    d2 = jnp.min(jnp.sum(diff * diff, axis=1))
    min_ref[0] = jnp.minimum(min_ref[0], d2)


def _combine_body(sc_min_ref, tc_min_ref, part_ref, out_ref):
    m = jnp.minimum(jnp.min(sc_min_ref[...]), tc_min_ref[0])
    out_ref[0] = part_ref[0] + 0.3 * jnp.exp(-jnp.sqrt(m))


def _sc_scan_body(sp_hbm, pat_hbm, out_hbm, pbuf, psum_v, zbuf, pat_v,
                  idx_v, bufs, a_v, stage_v, shared, sems, psem):
    cid = lax.axis_index("c")
    sid = lax.axis_index("s")
    wid = sid * 2 + cid
    base = wid * NPW
    lane = jax.lax.iota(jnp.int32, 16)

    def start(i, buf, sem):
        row0 = pl.multiple_of(base + i * CHUNK, 8)
        pltpu.make_async_copy(sp_hbm.at[pl.ds(row0, CHUNK)], buf, sem).start()

    def wait(buf, sem):
        pltpu.make_async_copy(sp_hbm.at[pl.ds(base, CHUNK)], buf, sem).wait()

    for b in range(NBUF):
        start(b, bufs[b], sems[b])

    prow0 = pl.multiple_of(sid * PROWS_PW, 8)
    pcopy = pltpu.make_async_copy(pat_hbm.at[pl.ds(prow0, PROWS_PW)], pbuf,
                                  psem)
    pcopy.start()
    for c in range(NCH):
        zbuf[c] = jnp.zeros((16,), jnp.float32)
    idx_v[pl.ds(0, 16)] = lane
    idx_v[pl.ds(16, 16)] = lane + 16

    @pl.when(sid == 0)
    def _zero():
        pltpu.sync_copy(zbuf, shared)

    pcopy.wait()
    plsc.subcore_barrier()
    for c in range(NCH):
        def psum_body(r, acc, c=c):
            return acc + pbuf[r, pl.ds(16 * c, 16)]

        acc = lax.fori_loop(1, PROWS_PW, psum_body,
                            pbuf[0, pl.ds(16 * c, 16)])
        psum_v[c] = acc
    pltpu.sync_copy(psum_v, shared.at[idx_v], add=True)
    plsc.subcore_barrier()
    pltpu.sync_copy(shared, pat_v)
    scale = jnp.float32(1.0 / PATROWS)
    pats = [pat_v[c] * scale for c in range(NCH)]

    def compute(buf, mins):
        def group_body(g, mins):
            def row_body(q, carry):
                for u in range(4):
                    r = q * 4 + u
                    row = g * 16 + r
                    d = buf[row, pl.ds(0, 16)] - pats[0]
                    acc = d * d
                    for c in range(1, NCH):
                        d = buf[row, pl.ds(16 * c, 16)] - pats[c]
                        acc = acc + d * d
                    a_v[r] = acc
                return carry

            lax.fori_loop(0, 4, row_body, 0)
            tot = plsc.load_gather(
                a_v, [lane, jnp.full((16,), 0, jnp.int32)])
            for c in range(1, 16):
                tot = tot + plsc.load_gather(
                    a_v, [lane, jnp.full((16,), c, jnp.int32)])
            return jnp.minimum(mins, tot)

        return lax.fori_loop(0, GROUPS, group_body, mins)

    mins0 = jnp.full((16,), jnp.inf, dtype=jnp.float32)

    def ring_body(jb, mins):
        for b in range(NBUF):
            i = jb * NBUF + b
            wait(bufs[b], sems[b])
            mins = compute(bufs[b], mins)

            @pl.when(i + NBUF < NCHUNKS)
            def _(i=i, b=b):
                start(i + NBUF, bufs[b], sems[b])

        return mins

    mins = lax.fori_loop(0, NCHUNKS // NBUF, ring_body, mins0)

    stage_v[...] = mins
    pltpu.sync_copy(stage_v, out_hbm.at[wid])


@functools.partial(
    pl.kernel,
    mesh=plsc.VectorSubcoreMesh(core_axis_name="c", subcore_axis_name="s"),
    out_type=jax.ShapeDtypeStruct((NWORKERS, 16), jnp.float32),
    compiler_params=pltpu.CompilerParams(needs_layout_passes=False),
    scratch_types=(
        [pltpu.VMEM((PROWS_PW, DIM), jnp.float32),
         pltpu.VMEM((NCH, 16), jnp.float32),
         pltpu.VMEM((NCH, 16), jnp.float32),
         pltpu.VMEM((NCH, 16), jnp.float32),
         pltpu.VMEM((NCH,), jnp.int32)]
        + [pltpu.VMEM((CHUNK, DIM), jnp.float32) for _ in range(NBUF)]
        + [pltpu.VMEM((16, 16), jnp.float32),
           pltpu.VMEM((16,), jnp.float32),
           pltpu.VMEM_SHARED((NCH, 16), jnp.float32)]
        + [pltpu.SemaphoreType.DMA for _ in range(NBUF + 1)]
    ),
)
def _sc_scan(sp_hbm, pat_hbm, out_hbm, pbuf, psum_v, zbuf, pat_v, idx_v,
             *rest):
    bufs = rest[:NBUF]
    a_v, stage_v, shared = rest[NBUF], rest[NBUF + 1], rest[NBUF + 2]
    sems = rest[NBUF + 3:NBUF + 3 + NBUF]
    psem = rest[NBUF + 3 + NBUF]
    _sc_scan_body(sp_hbm, pat_hbm, out_hbm, pbuf, psum_v, zbuf, pat_v,
                  idx_v, bufs, a_v, stage_v, shared, sems, psem)


def kernel(pattern, self_patterns, self_mean, self_var, W1, b1, W2, b2):
    sc_min = _sc_scan(self_patterns, pattern)

    part, tc_min = pl.pallas_call(
        _tc_scan_body,
        grid=(TC_NB,),
        in_specs=[
            pl.BlockSpec((PATROWS, DIM), lambda i: (0, 0)),
            pl.BlockSpec((TC_BLOCK, DIM), lambda i: (TC_BLOCK0 + i, 0)),
            pl.BlockSpec((1, DIM), lambda i: (0, 0)),
            pl.BlockSpec((1, DIM), lambda i: (0, 0)),
            pl.BlockSpec((DIM, 2 * DIM), lambda i: (0, 0)),
            pl.BlockSpec((1, DIM), lambda i: (0, 0)),
            pl.BlockSpec((1, DIM), lambda i: (0, 0)),
            pl.BlockSpec(memory_space=pltpu.SMEM),
        ],
        out_specs=[pl.BlockSpec(memory_space=pltpu.SMEM),
                   pl.BlockSpec(memory_space=pltpu.SMEM)],
        out_shape=[jax.ShapeDtypeStruct((1,), jnp.float32),
                   jax.ShapeDtypeStruct((1,), jnp.float32)],
        scratch_shapes=[pltpu.VMEM((1, DIM), jnp.float32)],
    )(pattern, self_patterns, self_mean.reshape(1, DIM),
      self_var.reshape(1, DIM), W1, b1.reshape(1, DIM), W2, b2)

    out = pl.pallas_call(
        _combine_body,
        in_specs=[
            pl.BlockSpec((NWORKERS, 16), lambda: (0, 0)),
            pl.BlockSpec(memory_space=pltpu.SMEM),
            pl.BlockSpec(memory_space=pltpu.SMEM),
        ],
        out_specs=pl.BlockSpec(memory_space=pltpu.SMEM),
        out_shape=jax.ShapeDtypeStruct((1,), jnp.float32),
    )(sc_min, tc_min, part)
    return out[0]
